# Initial kernel scaffold; baseline (speedup 1.0000x reference)
import jax
import jax.numpy as jnp
from jax.experimental import pallas as pl


def kernel(x, edge_index, edge_attr, batch, primary_label, W1, We):
    raise NotImplementedError("write your pallas kernel here")


# speedup vs baseline: 2.6418x; 2.6418x over previous
import functools

import jax
import jax.numpy as jnp
from jax import lax
from jax.experimental import pallas as pl
from jax.experimental.pallas import tpu as pltpu
from jax.experimental.pallas import tpu_sc as plsc

N = 10000
E = 320000
D = 128
DE = 16
DEP = 32
B = 128

NC = 2
NS = 16
NW = NC * NS
EPW = E // NW
K = 80
NCHUNK = EPW // K
N_PAD = 10240
RPT = N_PAD // NS

RB = 1000
NB = N // RB


def _sc_edge_agg(x, src, dst, emb, z128):
    mesh = plsc.VectorSubcoreMesh(core_axis_name="c", subcore_axis_name="s")

    @functools.partial(
        pl.kernel,
        mesh=mesh,
        out_type=jax.ShapeDtypeStruct((NC * N_PAD, D), jnp.float32),
        scratch_types=[
            pltpu.VMEM((K,), jnp.int32),
            pltpu.VMEM((K,), jnp.int32),
            pltpu.VMEM((K, D), jnp.float32),
            pltpu.VMEM((K, D), jnp.float32),
            pltpu.VMEM_SHARED((N_PAD, D), jnp.float32),
            pltpu.SemaphoreType.DMA,
        ],
    )
    def body(x_hbm, src_hbm, dst_hbm, emb_hbm, z128_hbm,
             aggx_out,
             srcv, dstv, rowsv, embv, aggsh, sem):
        cid = lax.axis_index("c")
        sid = lax.axis_index("s")
        wid = sid * NC + cid
        JJ = RPT // K

        iota16 = lax.broadcasted_iota(jnp.int32, (16,), 0)

        def set_own_rows(j):
            for q in range(K // 16):
                dstv[pl.ds(q * 16, 16)] = sid * RPT + j * K + q * 16 + iota16

        pltpu.sync_copy(z128_hbm, rowsv)
        for j in range(JJ):
            set_own_rows(j)
            pltpu.sync_copy(rowsv, aggsh.at[dstv])
        plsc.subcore_barrier()

        base0 = wid * EPW

        def chunk(c, carry):
            base = base0 + c * K
            pltpu.sync_copy(src_hbm.at[pl.ds(base, K)], srcv)
            pltpu.sync_copy(dst_hbm.at[pl.ds(base, K)], dstv)
            pltpu.sync_copy(emb_hbm.at[pl.ds(base, K)], embv)
            pltpu.async_copy(x_hbm.at[srcv], rowsv, sem).wait()
            pltpu.sync_copy(rowsv, aggsh.at[dstv], add=True)
            pltpu.sync_copy(embv, aggsh.at[dstv], add=True)
            return carry

        lax.fori_loop(0, NCHUNK, chunk, 0)
        plsc.subcore_barrier()

        out0 = cid * N_PAD + sid * RPT
        for j in range(JJ):
            set_own_rows(j)
            pltpu.sync_copy(aggsh.at[dstv], rowsv)
            pltpu.sync_copy(rowsv, aggx_out.at[pl.ds(out0 + j * K, K)])

    return body(x, src, dst, emb, z128)


EB = 8000


def _emb_body(ea_ref, we_ref, out_ref):
    out_ref[...] = lax.dot(ea_ref[...], we_ref[...],
                           precision=lax.Precision.HIGHEST)


def _tc_emb(edge_attr, We):
    return pl.pallas_call(
        _emb_body,
        grid=(E // EB,),
        in_specs=[
            pl.BlockSpec((EB, DE), lambda i: (i, 0)),
            pl.BlockSpec((DE, D), lambda i: (0, 0)),
        ],
        out_specs=pl.BlockSpec((EB, D), lambda i: (i, 0)),
        out_shape=jax.ShapeDtypeStruct((E, D), jnp.float32),
    )(edge_attr, We)


def _tc_body(x_ref, aggx_ref, b_ref, bn_ref, lab_ref,
             w1_ref, out_ref, accS, accA, cpool):
    i = pl.program_id(0)
    f32 = jnp.float32
    HI = lax.Precision.HIGHEST

    @pl.when(i == 0)
    def _init():
        accS[...] = jnp.zeros_like(accS)
        accA[...] = jnp.zeros_like(accA)

    @pl.when(i < NB)
    def _phase0():
        h = x_ref[...] + aggx_ref[0] + aggx_ref[1]
        nr = jnp.maximum(lax.dot(h, w1_ref[...], precision=HI), 0.0)
        out_ref[...] = nr

        bb = b_ref[0, 0, :]
        maskf = (lab_ref[0, 0, :] == -1).astype(f32)
        islf = (bb != bn_ref[0, 0, :]).astype(f32)
        oneT = (lax.broadcasted_iota(jnp.int32, (B, RB), 0)
                == bb[None, :]).astype(f32)
        accS[...] += lax.dot(oneT, nr * maskf[:, None], precision=HI)
        colid = lax.broadcasted_iota(jnp.int32, (RB, D), 1)
        aux = jnp.where(colid == 0, maskf[:, None],
                        jnp.where(colid == 1, (maskf * islf)[:, None], 0.0))
        accA[...] += lax.dot(oneT, aux, precision=HI)

    @pl.when(i == NB - 1)
    def _finalize():
        cnt = accA[:, 0:1]
        flg = accA[:, 1:2]
        cpool[...] = (accS[...] / jnp.maximum(cnt, 1.0)
                      * (flg > 0.0).astype(f32))

    @pl.when(i >= NB)
    def _phase1():
        bb = b_ref[0, 0, :]
        onehot = (bb[:, None]
                  == lax.broadcasted_iota(jnp.int32, (RB, B), 1)).astype(f32)
        out_ref[...] = lax.dot(onehot, cpool[...], precision=HI)


def _tc_head(x, aggx2, batch3, bn3, lab3, W1):
    return pl.pallas_call(
        _tc_body,
        grid=(2 * NB,),
        in_specs=[
            pl.BlockSpec((RB, D), lambda i: (jnp.minimum(i, NB - 1), 0)),
            pl.BlockSpec((NC, RB, D), lambda i: (0, jnp.minimum(i, NB - 1), 0)),
            pl.BlockSpec((1, 1, RB), lambda i: (i % NB, 0, 0)),
            pl.BlockSpec((1, 1, RB), lambda i: (jnp.minimum(i, NB - 1), 0, 0)),
            pl.BlockSpec((1, 1, RB), lambda i: (jnp.minimum(i, NB - 1), 0, 0)),
            pl.BlockSpec((D, D), lambda i: (0, 0)),
        ],
        out_specs=pl.BlockSpec((RB, D), lambda i: (i % NB, i // NB)),
        out_shape=jax.ShapeDtypeStruct((N, 2 * D), jnp.float32),
        scratch_shapes=[
            pltpu.VMEM((B, D), jnp.float32),
            pltpu.VMEM((B, D), jnp.float32),
            pltpu.VMEM((B, D), jnp.float32),
        ],
    )(x, aggx2, batch3, bn3, lab3, W1)


def kernel(x, edge_index, edge_attr, batch, primary_label, W1, We):
    src = edge_index[0]
    dst = edge_index[1]
    z128 = jnp.zeros((K, D), jnp.float32)

    emb = _tc_emb(edge_attr, We)
    aggx2 = _sc_edge_agg(x, src, dst, emb, z128)
    aggx2 = aggx2.reshape(NC, N_PAD, D)

    batch_next = jnp.concatenate(
        [batch[1:], jnp.full((1,), B, jnp.int32)])
    batch3 = batch.reshape(NB, 1, RB)
    bn3 = batch_next.reshape(NB, 1, RB)
    lab3 = primary_label.reshape(NB, 1, RB)

    return _tc_head(x, aggx2, batch3, bn3, lab3, W1)
